# baseline (speedup 1.0000x reference)
import jax
import jax.numpy as jnp
from jax import lax
from jax.experimental import pallas as pl
from jax.experimental.pallas import tpu as pltpu
from jax.experimental.pallas import tpu_sc as plsc

N = 320000
D = 128
Q = 32
NQ = 5
W = NQ * Q
S = 10000
SP = 10240

T = 8000
GRID1 = N // T

NC = 2
NS = 16
ROWS_W = N // (NC * NS)
CH = 1000
NCHUNK = ROWS_W // CH
IDXW = 125
IPC = CH // IDXW
SROWS = SP // NS
CH4 = 400
NCHUNK4 = ROWS_W // CH4

F32 = jnp.float32
BF16 = jnp.bfloat16


def _k1_body(x_ref, wcat_ref, bcat_ref, walt_ref, h_ref, aa_ref):
    xb = x_ref[...]
    y = jnp.dot(xb.astype(BF16), wcat_ref[...],
                preferred_element_type=F32)
    hv = xb + jnp.maximum(y[:, :D] + bcat_ref[:, :D], 0.0)
    za = jnp.maximum(y[:, D:] + bcat_ref[:, D:], 0.0)
    aa = jnp.sum((xb + za) * walt_ref[...], axis=1, keepdims=True)
    h_ref[...] = hv
    aa_ref[...] = aa


def _run_k1(x, Wv1, bv1, Wa1, ba1, Wal):
    wcat = jnp.concatenate([Wv1, Wa1], axis=1).astype(BF16)
    bcat = jnp.concatenate([bv1, ba1]).reshape(1, 2 * D)
    walt = Wal.reshape(1, D)
    return pl.pallas_call(
        _k1_body,
        grid=(GRID1,),
        in_specs=[
            pl.BlockSpec((T, D), lambda i: (i, 0)),
            pl.BlockSpec((D, 2 * D), lambda i: (0, 0)),
            pl.BlockSpec((1, 2 * D), lambda i: (0, 0)),
            pl.BlockSpec((1, D), lambda i: (0, 0)),
        ],
        out_specs=[
            pl.BlockSpec((T, D), lambda i: (i, 0)),
            pl.BlockSpec((T, 1), lambda i: (i, 0)),
        ],
        out_shape=[
            jax.ShapeDtypeStruct((N, D), F32),
            jax.ShapeDtypeStruct((N, 1), F32),
        ],
    )(x, wcat, bcat, walt)


def _k2_body(h_hbm, aa_hbm, idx_hbm, agg_hbm, hbuf, zbuf, idxbuf, aabuf, aggS):
    c = lax.axis_index("c")
    s = lax.axis_index("s")

    zero = jnp.zeros((16,), F32)

    def _zrow(buf):
        def body(r, carry):
            for j in range(Q // 16):
                buf[r, pl.ds(j * 16, 16)] = zero
            return carry
        return body

    lax.fori_loop(0, SROWS, _zrow(zbuf), 0)

    row0 = s * SROWS

    def _zero_agg():
        pltpu.sync_copy(zbuf, aggS.at[pl.ds(row0, SROWS)])

    _zero_agg()
    plsc.subcore_barrier()

    rbase = c * (N // NC) + s * ROWS_W

    def _scatter_chunk(base):
        irow = pl.multiple_of(base // IDXW, 8)
        pltpu.sync_copy(idx_hbm.at[pl.ds(irow, IPC)], idxbuf)
        for j in range(IPC):
            pltpu.sync_copy(hbuf.at[pl.ds(j * IDXW, IDXW)],
                            aggS.at[idxbuf.at[j]], add=True)

    for q in range(4):
        def _chunk(k, carry):
            base = rbase + k * CH
            pltpu.sync_copy(
                h_hbm.at[pl.ds(base, CH), pl.ds(q * Q, Q)], hbuf)
            _scatter_chunk(base)
            return carry

        lax.fori_loop(0, NCHUNK, _chunk, 0)
        plsc.subcore_barrier()

        pltpu.sync_copy(aggS.at[pl.ds(row0, SROWS)],
                        agg_hbm.at[c, q, pl.ds(row0, SROWS)])
        _zero_agg()
        plsc.subcore_barrier()

    lax.fori_loop(0, CH, _zrow(hbuf), 0)
    lanes = lax.iota(jnp.int32, 16)
    col0 = jnp.zeros((16,), jnp.int32)
    col1 = jnp.ones((16,), jnp.int32)
    ones16 = jnp.ones((16,), F32)

    def _setones(j, carry):
        plsc.store_scatter(hbuf, [j * 16 + lanes, col1], ones16)
        return carry

    lax.fori_loop(0, CH // 16, _setones, 0)

    def _chunk4(k, carry):
        base = rbase + k * CH
        pltpu.sync_copy(aa_hbm.at[pl.ds(base, CH)], aabuf)

        def _fill(j, c2):
            o = j * 16
            av = aabuf[pl.ds(o, 16)]
            plsc.store_scatter(hbuf, [o + lanes, col0], av)
            return c2

        lax.fori_loop(0, CH // 16, _fill, 0)
        _scatter_chunk(base)
        return carry

    lax.fori_loop(0, NCHUNK, _chunk4, 0)
    plsc.subcore_barrier()
    pltpu.sync_copy(aggS.at[pl.ds(row0, SROWS)],
                    agg_hbm.at[c, 4, pl.ds(row0, SROWS)])


def _run_k2(h, aa1, index):
    idx2 = index.reshape(N // IDXW, IDXW)
    mesh = plsc.VectorSubcoreMesh(core_axis_name="c", subcore_axis_name="s")
    fn = pl.kernel(
        _k2_body,
        out_type=jax.ShapeDtypeStruct((NC, NQ, SP, Q), F32),
        mesh=mesh,
        scratch_types=[
            pltpu.VMEM((CH, Q), F32),
            pltpu.VMEM((SROWS, Q), F32),
            pltpu.VMEM((IPC, IDXW), jnp.int32),
            pltpu.VMEM((CH,), F32),
            pltpu.VMEM_SHARED((SP, Q), F32),
        ],
        compiler_params=pltpu.CompilerParams(use_tc_tiling_on_sc=False,
                                             needs_layout_passes=False),
    )
    return fn(h, aa1, idx2)


def _k3_body(aggp_ref, w2_ref, b2_ref, wvlt_ref, bvl_ref, c_ref):
    a = jnp.concatenate(
        [aggp_ref[0, q] + aggp_ref[1, q] for q in range(4)], axis=1)
    h2 = a + jnp.maximum(jnp.dot(a, w2_ref[...], preferred_element_type=F32)
                         + b2_ref[...], 0.0)
    sv = jnp.sum(h2 * wvlt_ref[...], axis=1, keepdims=True) + bvl_ref[...]
    e = aggp_ref[0, 4] + aggp_ref[1, 4]
    mean = e[:, 0:1] / jnp.maximum(e[:, 1:2], 1.0)
    c_ref[...] = sv - mean


def _run_k3(agg_parts, Wv2, bv2, Wvl, bvl):
    TS = 512
    return pl.pallas_call(
        _k3_body,
        grid=(SP // TS,),
        in_specs=[
            pl.BlockSpec((NC, NQ, TS, Q), lambda i: (0, 0, i, 0)),
            pl.BlockSpec((D, D), lambda i: (0, 0)),
            pl.BlockSpec((1, D), lambda i: (0, 0)),
            pl.BlockSpec((1, D), lambda i: (0, 0)),
            pl.BlockSpec((1, 1), lambda i: (0, 0)),
        ],
        out_specs=pl.BlockSpec((TS, 1), lambda i: (i, 0)),
        out_shape=jax.ShapeDtypeStruct((SP, 1), F32),
    )(agg_parts, Wv2, bv2.reshape(1, D), Wvl.reshape(1, D),
      bvl.reshape(1, 1))


def _k4_body(c_hbm, aa_hbm, idx_hbm, out_hbm, cbuf, aabuf, idxbuf, obuf):
    c = lax.axis_index("c")
    s = lax.axis_index("s")
    pltpu.sync_copy(c_hbm, cbuf)
    rbase = c * (N // NC) + s * ROWS_W

    def _chunk(k, carry):
        base = rbase + k * CH4
        pltpu.sync_copy(aa_hbm.at[pl.ds(base, CH4)], aabuf)
        pltpu.sync_copy(idx_hbm.at[pl.ds(base, CH4)], idxbuf)

        def _inner(j, carry2):
            o = j * 16
            iv = idxbuf[pl.ds(o, 16)]
            cv = plsc.load_gather(cbuf, [iv])
            obuf[pl.ds(o, 16)] = aabuf[pl.ds(o, 16)] + cv
            return carry2

        lax.fori_loop(0, CH4 // 16, _inner, 0)
        pltpu.sync_copy(obuf, out_hbm.at[pl.ds(base, CH4)])
        return carry

    lax.fori_loop(0, NCHUNK4, _chunk, 0)


def _run_k4(cvec, aa1, index):
    mesh = plsc.VectorSubcoreMesh(core_axis_name="c", subcore_axis_name="s")
    fn = pl.kernel(
        _k4_body,
        out_type=jax.ShapeDtypeStruct((N,), F32),
        mesh=mesh,
        scratch_types=[
            pltpu.VMEM((SP,), F32),
            pltpu.VMEM((CH4,), F32),
            pltpu.VMEM((CH4,), jnp.int32),
            pltpu.VMEM((CH4,), F32),
        ],
        compiler_params=pltpu.CompilerParams(needs_layout_passes=False,
                                             use_tc_tiling_on_sc=False),
    )
    return fn(cvec, aa1, index)


def kernel(x, index, Wv1, bv1, Wv2, bv2, Wvl, bvl, Wa1, ba1, Wal, bal):
    h, aa = _run_k1(x, Wv1, bv1, Wa1, ba1, Wal)
    aa1 = aa.reshape(N)
    agg_parts = _run_k2(h, aa1, index)
    cvec = _run_k3(agg_parts, Wv2, bv2, Wvl, bvl).reshape(SP)
    out = _run_k4(cvec, aa1, index)
    return out, index
